# baseline (speedup 1.0000x reference)
import functools

import jax
import jax.numpy as jnp
from jax import lax
from jax.experimental import pallas as pl
from jax.experimental.pallas import tpu as pltpu
from jax.experimental.pallas import tpu_sc as plsc

_VOCAB = 100000
_BASIC_DIM = 300
_MODIF_DIM = 100
_OUT_DIM = _BASIC_DIM + _MODIF_DIM
_BATCH = 16384
_ALIGNED = 256
_TAIL = _BASIC_DIM - _ALIGNED
_LANE = 128

_NC = 2
_NS = 16
_NW = _NC * _NS
_B_PER_W = _BATCH // _NW
_CHUNK = 64
_NCHUNK = _B_PER_W // _CHUNK
_NBUF = 2


def _make_kernel():
    mesh = plsc.VectorSubcoreMesh(core_axis_name="c", subcore_axis_name="s")

    @functools.partial(
        pl.kernel,
        mesh=mesh,
        out_type=jax.ShapeDtypeStruct((_BATCH, _OUT_DIM), jnp.float32),
        compiler_params=pltpu.CompilerParams(
            use_tc_tiling_on_sc=True, needs_layout_passes=False),
        scratch_types=[
            pltpu.VMEM((_B_PER_W,), jnp.int32),
            pltpu.VMEM((_NBUF, _CHUNK, _LANE), jnp.float32),
            pltpu.VMEM((_NBUF, _CHUNK, _LANE), jnp.float32),
            pltpu.VMEM((_NBUF, _CHUNK, _OUT_DIM), jnp.float32),
            pltpu.SemaphoreType.DMA,
            pltpu.SemaphoreType.DMA,
        ],
    )
    def k(ids_hbm, basic_hbm, tail_hbm, modif_hbm, out_hbm,
          idx_v, buf_t, buf_m, buf_c, sem, sem2):
        wid = lax.axis_index("s") * _NC + lax.axis_index("c")
        base = wid * _B_PER_W
        pltpu.sync_copy(ids_hbm.at[pl.ds(base, _B_PER_W)], idx_v)

        def fire(c):
            b = c % _NBUF
            cidx = idx_v.at[pl.ds(c * _CHUNK, _CHUNK)]
            pltpu.async_copy(
                basic_hbm.at[cidx, pl.ds(0, _ALIGNED)],
                buf_c.at[b, :, pl.ds(0, _ALIGNED)], sem)
            pltpu.async_copy(tail_hbm.at[cidx], buf_t.at[b], sem2)
            pltpu.async_copy(modif_hbm.at[cidx], buf_m.at[b], sem2)

        def drain(c):
            b = c % _NBUF
            pltpu.make_async_copy(
                basic_hbm.at[pl.ds(0, _CHUNK), pl.ds(0, _ALIGNED)],
                buf_c.at[b, :, pl.ds(0, _ALIGNED)], sem).wait()
            pltpu.make_async_copy(
                tail_hbm.at[pl.ds(0, _CHUNK)], buf_t.at[b], sem2).wait()
            pltpu.make_async_copy(
                modif_hbm.at[pl.ds(0, _CHUNK)], buf_m.at[b], sem2).wait()

        def finish(c):
            b = c % _NBUF

            def assemble(r, carry2):
                for dst, src in ((256, 0), (272, 16), (284, 28)):
                    buf_c[b, r, pl.ds(dst, 16)] = buf_t[b, r, pl.ds(src, 16)]
                for j in range(5):
                    buf_c[b, r, pl.ds(300 + 16 * j, 16)] = (
                        buf_m[b, r, pl.ds(16 * j, 16)])
                buf_c[b, r, pl.ds(368, 16)] = buf_m[b, r, pl.ds(68, 16)]
                buf_c[b, r, pl.ds(384, 16)] = buf_m[b, r, pl.ds(84, 16)]
                return carry2

            lax.fori_loop(0, _CHUNK, assemble, 0)
            pltpu.sync_copy(
                buf_c.at[b], out_hbm.at[pl.ds(base + c * _CHUNK, _CHUNK)])

        fire(0)

        def steady(i, carry):
            drain(i)
            fire(i + 1)
            finish(i)
            return carry

        lax.fori_loop(0, _NCHUNK - 1, steady, 0)
        drain(_NCHUNK - 1)
        finish(_NCHUNK - 1)

    return k


_kernel_call = _make_kernel()


def kernel(word_ids, basic, modif):
    basic_tail = jnp.pad(basic[:, _ALIGNED:], ((0, 0), (0, _LANE - _TAIL)))
    modif_pad = jnp.pad(modif, ((0, 0), (0, _LANE - _MODIF_DIM)))
    return _kernel_call(
        word_ids.astype(jnp.int32), basic, basic_tail, modif_pad)


# speedup vs baseline: 1.2530x; 1.2530x over previous
import functools

import jax
import jax.numpy as jnp
from jax import lax
from jax.experimental import pallas as pl
from jax.experimental.pallas import tpu as pltpu
from jax.experimental.pallas import tpu_sc as plsc

_VOCAB = 100000
_BASIC_DIM = 300
_MODIF_DIM = 100
_OUT_DIM = _BASIC_DIM + _MODIF_DIM
_BATCH = 16384
_ALIGNED = 256
_TAIL = _BASIC_DIM - _ALIGNED

_NC = 2
_NS = 16
_NW = _NC * _NS
_B_PER_W = _BATCH // _NW
_CHUNK = 64
_NCHUNK = _B_PER_W // _CHUNK
_NBUF = 2


def _make_kernel():
    mesh = plsc.VectorSubcoreMesh(core_axis_name="c", subcore_axis_name="s")

    @functools.partial(
        pl.kernel,
        mesh=mesh,
        out_type=jax.ShapeDtypeStruct((_BATCH, _OUT_DIM), jnp.float32),
        compiler_params=pltpu.CompilerParams(
            use_tc_tiling_on_sc=True, needs_layout_passes=False),
        scratch_types=[
            pltpu.VMEM((_B_PER_W,), jnp.int32),
            pltpu.VMEM((_NBUF, _CHUNK, _TAIL), jnp.float32),
            pltpu.VMEM((_NBUF, _CHUNK, _MODIF_DIM), jnp.float32),
            pltpu.VMEM((_NBUF, _CHUNK, _OUT_DIM), jnp.float32),
            pltpu.SemaphoreType.DMA,
            pltpu.SemaphoreType.DMA,
            pltpu.SemaphoreType.DMA,
            pltpu.SemaphoreType.DMA,
        ],
    )
    def k(ids_hbm, basic_hbm, modif_hbm, out_hbm,
          idx_v, buf_t, buf_m, buf_c, sem_a0, sem_a1, sem_t0, sem_t1):
        wid = lax.axis_index("s") * _NC + lax.axis_index("c")
        base = wid * _B_PER_W
        pltpu.sync_copy(ids_hbm.at[pl.ds(base, _B_PER_W)], idx_v)
        sem_a = (sem_a0, sem_a1)
        sem_t = (sem_t0, sem_t1)

        def fire(c, b):
            pltpu.async_copy(
                basic_hbm.at[idx_v.at[pl.ds(c * _CHUNK, _CHUNK)],
                             pl.ds(0, _ALIGNED)],
                buf_c.at[b, :, pl.ds(0, _ALIGNED)], sem_a[b])
            for t in range(_CHUNK // 16):
                vec = idx_v[pl.ds(c * _CHUNK + t * 16, 16)]
                for j in range(16):
                    r = t * 16 + j
                    wi = vec[j]
                    pltpu.async_copy(
                        basic_hbm.at[wi, pl.ds(_ALIGNED, _TAIL)],
                        buf_t.at[b, r], sem_t[b])
                    pltpu.async_copy(
                        modif_hbm.at[wi], buf_m.at[b, r], sem_t[b])

        def drain(b):
            pltpu.make_async_copy(
                basic_hbm.at[pl.ds(0, _CHUNK), pl.ds(0, _ALIGNED)],
                buf_c.at[b, :, pl.ds(0, _ALIGNED)], sem_a[b]).wait()
            for r in range(_CHUNK):
                pltpu.make_async_copy(
                    basic_hbm.at[0, pl.ds(_ALIGNED, _TAIL)],
                    buf_t.at[b, r], sem_t[b]).wait()
                pltpu.make_async_copy(
                    modif_hbm.at[0], buf_m.at[b, r], sem_t[b]).wait()

        def finish(c, b):
            def assemble(r, carry2):
                for dst, src in ((256, 0), (272, 16), (284, 28)):
                    buf_c[b, r, pl.ds(dst, 16)] = buf_t[b, r, pl.ds(src, 16)]
                for j in range(5):
                    buf_c[b, r, pl.ds(300 + 16 * j, 16)] = (
                        buf_m[b, r, pl.ds(16 * j, 16)])
                buf_c[b, r, pl.ds(368, 16)] = buf_m[b, r, pl.ds(68, 16)]
                buf_c[b, r, pl.ds(384, 16)] = buf_m[b, r, pl.ds(84, 16)]
                return carry2

            lax.fori_loop(0, _CHUNK, assemble, 0)
            pltpu.sync_copy(
                buf_c.at[b], out_hbm.at[pl.ds(base + c * _CHUNK, _CHUNK)])

        fire(0, 0)

        def steady(i, carry):
            c0 = 2 * i
            fire(c0 + 1, 1)
            drain(0)
            finish(c0, 0)
            fire(c0 + 2, 0)
            drain(1)
            finish(c0 + 1, 1)
            return carry

        lax.fori_loop(0, _NCHUNK // 2 - 1, steady, 0)
        c0 = _NCHUNK - 2
        fire(c0 + 1, 1)
        drain(0)
        finish(c0, 0)
        drain(1)
        finish(c0 + 1, 1)

    return k


_kernel_call = _make_kernel()


def kernel(word_ids, basic, modif):
    return _kernel_call(word_ids.astype(jnp.int32), basic, modif)
